# baseline (speedup 1.0000x reference)
import functools

import jax
import jax.numpy as jnp
from jax.experimental import pallas as pl
from jax.experimental.pallas import tpu as pltpu


def _mm(a, b):
    return jax.lax.dot_general(
        a, b, (((1,), (0,)), ((), ())),
        preferred_element_type=jnp.float32,
        precision=jax.lax.Precision.DEFAULT)


def _dense_body(x_ref, w_ref, o_ref):
    o_ref[...] = _mm(x_ref[...], w_ref[...]).astype(jnp.bfloat16)


def _dense_bf16(x, w):
    return pl.pallas_call(
        _dense_body,
        out_shape=jax.ShapeDtypeStruct((x.shape[0], w.shape[1]),
                                       jnp.bfloat16),
    )(x, w)


def _l1_body(adj_ref, s1_ref, b1_ref, w2_ref,
             x1_ref, x2p_ref, s_ref, s2scr,
             *, bi, bc, h1, n, npad, ni):
    i = pl.program_id(0)

    @pl.when(i == 0)
    def _():
        s_ref[...] = jnp.zeros_like(s_ref)
        s_ref[pl.ds(0, n), :h1] = s1_ref[...]
        s2scr[...] = jnp.zeros_like(s2scr)

    cur_b = (i * bi) // bc
    prev_b = ((i - 1) * bi) // bc

    @pl.when((i > 0) & (cur_b > prev_b))
    def _():
        s_ref[pl.ds((cur_b - 1) * bc, bc), h1:] = \
            s2scr[pl.ds((cur_b - 1) * bc, bc), :]

    out = _mm(adj_ref[...], s_ref[pl.ds(0, n), :])
    x1 = jnp.maximum(out[:, :h1] + b1_ref[...], 0.0)
    x1_ref[...] = x1
    x2p_ref[...] = out[:, h1:]
    s2scr[pl.ds(i * bi, bi), :] = _mm(x1, w2_ref[...]).astype(jnp.bfloat16)

    tail = (((ni - 1) * bi) // bc) * bc

    @pl.when(i == ni - 1)
    def _():
        s_ref[pl.ds(tail, npad - tail), h1:] = \
            s2scr[pl.ds(tail, npad - tail), :]


def _l2_body(adj_ref, s_ref, x2p_ref, b2_ref, x2_ref,
             *, bi, bc, h1, nk, valid_last):
    i = pl.program_id(0)
    k = pl.program_id(1)
    kb = (i * bi) // bc

    @pl.when(k == kb)
    def _():
        x2_ref[...] = x2p_ref[...] + b2_ref[...]

    @pl.when((k >= kb) & (k < nk - 1))
    def _():
        x2_ref[...] = x2_ref[...] + _mm(
            adj_ref[...], s_ref[pl.ds(k * bc, bc), h1:])

    @pl.when(k == nk - 1)
    def _():
        blk = adj_ref[...]
        if valid_last != bc:
            col = jax.lax.broadcasted_iota(jnp.int32, blk.shape, 1)
            blk = jnp.where(col < valid_last, blk, 0.0)
        x2_ref[...] = x2_ref[...] + _mm(
            blk, s_ref[pl.ds(k * bc, bc), h1:])


def gcn2(x, adj, W1, b1, W2, b2, bi=400, bc=1024):
    n = adj.shape[0]
    h1 = W1.shape[1]
    h2 = W2.shape[1]
    ni = n // bi
    nk = -(-n // bc)
    npad = nk * bc
    valid_last = n - (nk - 1) * bc

    s1 = _dense_bf16(x, W1)
    w2_bf = W2.astype(jnp.bfloat16)

    x1, x2p, s_buf = pl.pallas_call(
        functools.partial(_l1_body, bi=bi, bc=bc, h1=h1, n=n, npad=npad,
                          ni=ni),
        grid=(ni,),
        in_specs=[
            pl.BlockSpec((bi, n), lambda i: (i, 0)),
            pl.BlockSpec((n, h1), lambda i: (0, 0)),
            pl.BlockSpec((1, h1), lambda i: (0, 0)),
            pl.BlockSpec((h1, h2), lambda i: (0, 0)),
        ],
        out_specs=[
            pl.BlockSpec((bi, h1), lambda i: (i, 0)),
            pl.BlockSpec((bi, h2), lambda i: (i, 0)),
            pl.BlockSpec((npad, h1 + h2), lambda i: (0, 0)),
        ],
        out_shape=[
            jax.ShapeDtypeStruct((n, h1), jnp.float32),
            jax.ShapeDtypeStruct((n, h2), jnp.float32),
            jax.ShapeDtypeStruct((npad, h1 + h2), jnp.bfloat16),
        ],
        scratch_shapes=[pltpu.VMEM((npad, h2), jnp.bfloat16)],
        compiler_params=pltpu.CompilerParams(
            dimension_semantics=("arbitrary",)
        ),
    )(adj, s1, b1.reshape(1, -1), w2_bf)

    x2 = pl.pallas_call(
        functools.partial(_l2_body, bi=bi, bc=bc, h1=h1, nk=nk,
                          valid_last=valid_last),
        grid=(ni, nk),
        in_specs=[
            pl.BlockSpec((bi, bc),
                         lambda i, k: (i, jnp.maximum(k, (i * bi) // bc))),
            pl.BlockSpec((npad, h1 + h2), lambda i, k: (0, 0)),
            pl.BlockSpec((bi, h2), lambda i, k: (i, 0)),
            pl.BlockSpec((1, h2), lambda i, k: (0, 0)),
        ],
        out_specs=pl.BlockSpec((bi, h2), lambda i, k: (i, 0)),
        out_shape=jax.ShapeDtypeStruct((n, h2), jnp.float32),
        compiler_params=pltpu.CompilerParams(
            dimension_semantics=("arbitrary", "arbitrary")
        ),
    )(adj, s_buf, x2p, b2.reshape(1, -1))

    return (x1, x2)


def kernel(x, adj, W1, b1, W2, b2):
    return gcn2(x, adj, W1, b1, W2, b2, bi=400, bc=1024)


# speedup vs baseline: 1.3234x; 1.3234x over previous
import functools

import jax
import jax.numpy as jnp
from jax.experimental import pallas as pl
from jax.experimental.pallas import tpu as pltpu


def _mm(a, b):
    return jax.lax.dot_general(
        a, b, (((1,), (0,)), ((), ())),
        preferred_element_type=jnp.float32,
        precision=jax.lax.Precision.DEFAULT)


def _dense_body(x_ref, w_ref, o_ref):
    o_ref[...] = _mm(x_ref[...], w_ref[...]).astype(jnp.bfloat16)


def _dense_bf16(x, w):
    return pl.pallas_call(
        _dense_body,
        out_shape=jax.ShapeDtypeStruct((x.shape[0], w.shape[1]),
                                       jnp.bfloat16),
    )(x, w)


def _l1_body(adj_ref, s1_ref, b1_ref, w2_ref,
             x1_ref, x2p_ref, s_ref, s2scr,
             *, bi, bc, gsz, h1, n, npad, ni):
    i = pl.program_id(0)

    @pl.when(i == 0)
    def _():
        s_ref[...] = jnp.zeros_like(s_ref)
        s_ref[pl.ds(0, n), :h1] = s1_ref[...]
        s2scr[...] = jnp.zeros_like(s2scr)

    g = i // gsz
    new_cnt = (g * gsz * bi) // bc
    prev_cnt = ((g - 1) * gsz * bi) // bc
    at_boundary = (i > 0) & (i % gsz == 0)

    @pl.when(at_boundary & (prev_cnt < new_cnt))
    def _():
        s_ref[pl.ds(prev_cnt * bc, bc), h1:] = \
            s2scr[pl.ds(prev_cnt * bc, bc), :]

    @pl.when(at_boundary & (prev_cnt + 1 < new_cnt))
    def _():
        s_ref[pl.ds((prev_cnt + 1) * bc, bc), h1:] = \
            s2scr[pl.ds((prev_cnt + 1) * bc, bc), :]

    out = _mm(adj_ref[...], s_ref[pl.ds(0, n), :])
    x1 = jnp.maximum(out[:, :h1] + b1_ref[...], 0.0)
    x1_ref[...] = x1
    x2p_ref[...] = out[:, h1:]
    s2scr[pl.ds(i * bi, bi), :] = _mm(x1, w2_ref[...]).astype(jnp.bfloat16)

    tail = ((((ni - 1) // gsz) * gsz * bi) // bc) * bc

    @pl.when(i == ni - 1)
    def _():
        s_ref[pl.ds(tail, npad - tail), h1:] = \
            s2scr[pl.ds(tail, npad - tail), :]


def _l2_body(adj_ref, s_ref, x2p_ref, b2_ref, x2_ref,
             *, bg, bc, h1, nk, valid_last):
    i = pl.program_id(0)
    k = pl.program_id(1)
    kb = (i * bg) // bc

    @pl.when(k == kb)
    def _():
        x2_ref[...] = x2p_ref[...] + b2_ref[...]

    @pl.when((k >= kb) & (k < nk - 1))
    def _():
        x2_ref[...] = x2_ref[...] + _mm(
            adj_ref[...], s_ref[pl.ds(k * bc, bc), h1:])

    @pl.when(k == nk - 1)
    def _():
        blk = adj_ref[...]
        if valid_last != bc:
            col = jax.lax.broadcasted_iota(jnp.int32, blk.shape, 1)
            blk = jnp.where(col < valid_last, blk, 0.0)
        x2_ref[...] = x2_ref[...] + _mm(
            blk, s_ref[pl.ds(k * bc, bc), h1:])


def gcn2(x, adj, W1, b1, W2, b2, bi=400, bc=1280, gsz=5):
    n = adj.shape[0]
    h1 = W1.shape[1]
    h2 = W2.shape[1]
    ni = n // bi
    nk = -(-n // bc)
    npad = nk * bc
    valid_last = n - (nk - 1) * bc
    bg = gsz * bi
    ng = n // bg

    s1 = _dense_bf16(x, W1)
    w2_bf = W2.astype(jnp.bfloat16)

    x1, x2p, s_buf = pl.pallas_call(
        functools.partial(_l1_body, bi=bi, bc=bc, gsz=gsz, h1=h1, n=n,
                          npad=npad, ni=ni),
        grid=(ni,),
        in_specs=[
            pl.BlockSpec((bi, n), lambda i: (i, 0)),
            pl.BlockSpec((n, h1), lambda i: (0, 0)),
            pl.BlockSpec((1, h1), lambda i: (0, 0)),
            pl.BlockSpec((h1, h2), lambda i: (0, 0)),
        ],
        out_specs=[
            pl.BlockSpec((bi, h1), lambda i: (i, 0)),
            pl.BlockSpec((bi, h2), lambda i: (i, 0)),
            pl.BlockSpec((npad, h1 + h2), lambda i: (0, 0)),
        ],
        out_shape=[
            jax.ShapeDtypeStruct((n, h1), jnp.float32),
            jax.ShapeDtypeStruct((n, h2), jnp.float32),
            jax.ShapeDtypeStruct((npad, h1 + h2), jnp.bfloat16),
        ],
        scratch_shapes=[pltpu.VMEM((npad, h2), jnp.bfloat16)],
        compiler_params=pltpu.CompilerParams(
            dimension_semantics=("arbitrary",)
        ),
    )(adj, s1, b1.reshape(1, -1), w2_bf)

    x2 = pl.pallas_call(
        functools.partial(_l2_body, bg=bg, bc=bc, h1=h1, nk=nk,
                          valid_last=valid_last),
        grid=(ng, nk),
        in_specs=[
            pl.BlockSpec((bg, bc),
                         lambda i, k: (i, jnp.maximum(k, (i * bg) // bc))),
            pl.BlockSpec((npad, h1 + h2), lambda i, k: (0, 0)),
            pl.BlockSpec((bg, h2), lambda i, k: (i, 0)),
            pl.BlockSpec((1, h2), lambda i, k: (0, 0)),
        ],
        out_specs=pl.BlockSpec((bg, h2), lambda i, k: (i, 0)),
        out_shape=jax.ShapeDtypeStruct((n, h2), jnp.float32),
        compiler_params=pltpu.CompilerParams(
            dimension_semantics=("arbitrary", "arbitrary")
        ),
    )(adj, s_buf, x2p, b2.reshape(1, -1))

    return (x1, x2)


def kernel(x, adj, W1, b1, W2, b2):
    return gcn2(x, adj, W1, b1, W2, b2, bi=400, bc=1280, gsz=5)
